# baseline (speedup 1.0000x reference)
import functools

import jax
import jax.numpy as jnp
from jax.experimental import pallas as pl
from jax.experimental.pallas import tpu as pltpu

CHUNK = 256
NBUF = 2


def _router_kernel(n_chunks, x_hbm, w_ref, b_ref, out_hbm, cnt_ref, acc_ref):
    acc_ref[...] = jnp.zeros_like(acc_ref)

    def chunk_body(x_blk, out_blk):
        acc = jnp.dot(x_blk[...], w_ref[...],
                      preferred_element_type=jnp.float32)
        logits = jnp.maximum(acc + b_ref[...], 0.0)
        out_blk[...] = logits
        nz = jnp.sum((logits > 0.0).astype(jnp.float32))
        acc_ref[...] += jnp.full(acc_ref.shape, nz, dtype=jnp.float32)

    pipeline = pltpu.emit_pipeline(
        chunk_body,
        grid=(n_chunks,),
        in_specs=[
            pl.BlockSpec((CHUNK, x_hbm.shape[1]), lambda i: (i, 0),
                         pipeline_mode=pl.Buffered(buffer_count=NBUF)),
        ],
        out_specs=[
            pl.BlockSpec((CHUNK, out_hbm.shape[1]), lambda i: (i, 0)),
        ],
    )
    pipeline(x_hbm, out_hbm)
    cnt_ref[...] = acc_ref[...]


@jax.jit
def _run(x, W, b):
    n_tokens, d_model = x.shape
    n_experts = W.shape[1]
    n_chunks = n_tokens // CHUNK
    b2 = b.reshape(1, n_experts)

    logits, counts = pl.pallas_call(
        functools.partial(_router_kernel, n_chunks),
        in_specs=[
            pl.BlockSpec(memory_space=pl.ANY),
            pl.BlockSpec(memory_space=pltpu.VMEM),
            pl.BlockSpec(memory_space=pltpu.VMEM),
        ],
        out_specs=[
            pl.BlockSpec(memory_space=pl.ANY),
            pl.BlockSpec(memory_space=pltpu.VMEM),
        ],
        out_shape=[
            jax.ShapeDtypeStruct((n_tokens, n_experts), jnp.float32),
            jax.ShapeDtypeStruct((8, 128), jnp.float32),
        ],
        scratch_shapes=[
            pltpu.VMEM((8, 128), jnp.float32),
        ],
        compiler_params=pltpu.CompilerParams(
            vmem_limit_bytes=110 * 1024 * 1024,
        ),
    )(x, W, b2)

    density = counts[0, 0] / (n_tokens * n_experts)
    return logits, density.astype(jnp.float32)


def kernel(x, W, b):
    return _run(x, W, b)


# speedup vs baseline: 1.2465x; 1.2465x over previous
import functools

import jax
import jax.numpy as jnp
from jax.experimental import pallas as pl
from jax.experimental.pallas import tpu as pltpu

CHUNK = 256
NBUF = 4


def _router_kernel(n_chunks, x_hbm, w_ref, b_ref, out_hbm, cnt_ref, acc_ref):
    acc_ref[...] = jnp.zeros_like(acc_ref)

    def chunk_body(x_blk, out_blk):
        acc = jnp.dot(x_blk[...], w_ref[...],
                      preferred_element_type=jnp.float32)
        logits = jnp.maximum(acc + b_ref[...], 0.0)
        out_blk[...] = logits
        nz = jnp.sum((logits > 0.0).astype(jnp.float32), axis=0,
                     keepdims=True)
        acc_ref[0:1, 0:nz.shape[1]] += nz

    pipeline = pltpu.emit_pipeline(
        chunk_body,
        grid=(n_chunks,),
        in_specs=[
            pl.BlockSpec((CHUNK, x_hbm.shape[1]), lambda i: (i, 0),
                         pipeline_mode=pl.Buffered(buffer_count=NBUF)),
        ],
        out_specs=[
            pl.BlockSpec((CHUNK, out_hbm.shape[1]), lambda i: (i, 0)),
        ],
    )
    pipeline(x_hbm, out_hbm)
    cnt_ref[...] = acc_ref[...]


@jax.jit
def _run(x, W, b):
    n_tokens, d_model = x.shape
    n_experts = W.shape[1]
    n_chunks = n_tokens // CHUNK
    b2 = b.reshape(1, n_experts)

    logits, counts = pl.pallas_call(
        functools.partial(_router_kernel, n_chunks),
        in_specs=[
            pl.BlockSpec(memory_space=pl.ANY),
            pl.BlockSpec(memory_space=pltpu.VMEM),
            pl.BlockSpec(memory_space=pltpu.VMEM),
        ],
        out_specs=[
            pl.BlockSpec(memory_space=pl.ANY),
            pl.BlockSpec(memory_space=pltpu.VMEM),
        ],
        out_shape=[
            jax.ShapeDtypeStruct((n_tokens, n_experts), jnp.float32),
            jax.ShapeDtypeStruct((8, 128), jnp.float32),
        ],
        scratch_shapes=[
            pltpu.VMEM((8, 128), jnp.float32),
        ],
        compiler_params=pltpu.CompilerParams(
            vmem_limit_bytes=110 * 1024 * 1024,
        ),
    )(x, W, b2)

    density = jnp.sum(counts[0, :n_experts]) / (n_tokens * n_experts)
    return logits, density.astype(jnp.float32)


def kernel(x, W, b):
    return _run(x, W, b)


# speedup vs baseline: 1.2493x; 1.0022x over previous
import functools

import jax
import jax.numpy as jnp
from jax.experimental import pallas as pl
from jax.experimental.pallas import tpu as pltpu

CHUNK = 256
NBUF = 3


def _router_kernel(n_chunks, x_hbm, w_ref, b_ref, out_hbm, cnt_ref, acc_ref):
    acc_ref[...] = jnp.zeros_like(acc_ref)

    def chunk_body(x_blk, out_blk):
        acc = jnp.dot(x_blk[...], w_ref[...],
                      preferred_element_type=jnp.float32)
        logits = jnp.maximum(acc + b_ref[...], 0.0)
        out_blk[...] = logits
        nz = jnp.sum((logits > 0.0).astype(jnp.float32), axis=0,
                     keepdims=True)
        acc_ref[0:1, 0:nz.shape[1]] += nz

    pipeline = pltpu.emit_pipeline(
        chunk_body,
        grid=(n_chunks,),
        in_specs=[
            pl.BlockSpec((CHUNK, x_hbm.shape[1]), lambda i: (i, 0),
                         pipeline_mode=pl.Buffered(buffer_count=NBUF)),
        ],
        out_specs=[
            pl.BlockSpec((CHUNK, out_hbm.shape[1]), lambda i: (i, 0)),
        ],
    )
    pipeline(x_hbm, out_hbm)
    cnt_ref[...] = acc_ref[...]


@jax.jit
def _run(x, W, b):
    n_tokens, d_model = x.shape
    n_experts = W.shape[1]
    n_chunks = n_tokens // CHUNK
    b2 = b.reshape(1, n_experts)

    logits, counts = pl.pallas_call(
        functools.partial(_router_kernel, n_chunks),
        in_specs=[
            pl.BlockSpec(memory_space=pl.ANY),
            pl.BlockSpec(memory_space=pltpu.VMEM),
            pl.BlockSpec(memory_space=pltpu.VMEM),
        ],
        out_specs=[
            pl.BlockSpec(memory_space=pl.ANY),
            pl.BlockSpec(memory_space=pltpu.VMEM),
        ],
        out_shape=[
            jax.ShapeDtypeStruct((n_tokens, n_experts), jnp.float32),
            jax.ShapeDtypeStruct((8, 128), jnp.float32),
        ],
        scratch_shapes=[
            pltpu.VMEM((8, 128), jnp.float32),
        ],
        compiler_params=pltpu.CompilerParams(
            vmem_limit_bytes=110 * 1024 * 1024,
        ),
    )(x, W, b2)

    density = jnp.sum(counts[0, :n_experts]) / (n_tokens * n_experts)
    return logits, density.astype(jnp.float32)


def kernel(x, W, b):
    return _run(x, W, b)
